# baseline (speedup 1.0000x reference)
import functools

import jax
import jax.numpy as jnp
from jax import lax
from jax.experimental import pallas as pl
from jax.experimental.pallas import tpu as pltpu
from jax.experimental.pallas import tpu_sc as plsc

_NUSER = 5000
_NITEM = 5000
_NN = 10000
_NNP = 10240
_DC = 128
_DF = 512
_NCLS = 32
_NROUT = 3
_TDECAY = 0.5
_BATCH = 1024

_NWORK = 32
_ACCR = 1920
_TRASH = 1904
_KCH = 80
_STRIPE = _NNP // 16

_NPASS_R = -(-_NUSER // _TRASH)
_NPASS_F = -(-_NN // _TRASH)

_f32 = jnp.float32
_i32 = jnp.int32


def _l2n(x):
    return x / jnp.maximum(jnp.sqrt(jnp.sum(x * x, axis=-1, keepdims=True)), 1e-12)


def _leaky(x, s):
    return jnp.where(x >= 0, x, s * x)


_GD = lax.GatherDimensionNumbers(offset_dims=(), collapsed_slice_dims=(0,),
                                 start_index_map=(0,))


def _vbcast(v, t):
    idx = jnp.full((16, 1), t, _i32)
    return lax.gather(v, idx, _GD, slice_sizes=(1,),
                      mode=lax.GatherScatterMode.PROMISE_IN_BOUNDS)


def _pre_body(feat, Ws, bs, prefr, Wg, a2, f_o, pref_o, hi_o, hu_o, sdi_o, sdu_o):
    ft = jnp.dot(feat[...], Ws[...], preferred_element_type=_f32) + bs[...]
    f = _l2n(_leaky(ft, 0.01))
    p = _l2n(prefr[...])
    hi = jnp.dot(f, Wg[...], preferred_element_type=_f32)
    hu = jnp.dot(p, Wg[...], preferred_element_type=_f32)
    f_o[...] = f
    pref_o[...] = p
    hi_o[...] = hi
    hu_o[...] = hu
    sdi_o[...] = jnp.dot(hi, a2[...], preferred_element_type=_f32)
    sdu_o[...] = jnp.dot(hu, a2[...], preferred_element_type=_f32)


def _pre_call(features, Ws, bs2, preference, Wg, a2):
    br = 1024
    sds = jax.ShapeDtypeStruct
    return pl.pallas_call(
        _pre_body,
        grid=(5,),
        in_specs=[
            pl.BlockSpec((br, _DF), lambda i: (i, 0)),
            pl.BlockSpec((_DF, _DC), lambda i: (0, 0)),
            pl.BlockSpec((1, _DC), lambda i: (0, 0)),
            pl.BlockSpec((br, _DC), lambda i: (i, 0)),
            pl.BlockSpec((_DC, _DC), lambda i: (0, 0)),
            pl.BlockSpec((_DC, 8), lambda i: (0, 0)),
        ],
        out_specs=[
            pl.BlockSpec((br, _DC), lambda i: (i, 0)),
            pl.BlockSpec((br, _DC), lambda i: (i, 0)),
            pl.BlockSpec((br, _DC), lambda i: (i, 0)),
            pl.BlockSpec((br, _DC), lambda i: (i, 0)),
            pl.BlockSpec((br, 8), lambda i: (i, 0)),
            pl.BlockSpec((br, 8), lambda i: (i, 0)),
        ],
        out_shape=[
            sds((_NITEM, _DC), _f32),
            sds((_NUSER, _DC), _f32),
            sds((_NITEM, _DC), _f32),
            sds((_NUSER, _DC), _f32),
            sds((_NITEM, 8), _f32),
            sds((_NUSER, 8), _f32),
        ],
    )(features, Ws, bs2, preference, Wg, a2)


def _rout_body(pref, outp, denp, Wg, a2, pref_o, hu_o, sdu_o):
    o = outp[0] + outp[1]
    ones = jnp.ones((denp.shape[0], 1), _f32)
    dn = lax.dot_general(denp[...], ones, (((0,), (0,)), ((), ())),
                         preferred_element_type=_f32)
    p = _l2n(pref[...] + o / (dn + 1e-16))
    hu = jnp.dot(p, Wg[...], preferred_element_type=_f32)
    pref_o[...] = p
    hu_o[...] = hu
    sdu_o[...] = jnp.dot(hu, a2[...], preferred_element_type=_f32)


def _rout_call(pref, outp, denp, Wg, a2):
    br = 1024
    sds = jax.ShapeDtypeStruct
    return pl.pallas_call(
        _rout_body,
        grid=(5,),
        in_specs=[
            pl.BlockSpec((br, _DC), lambda i: (i, 0)),
            pl.BlockSpec((2, br, _DC), lambda i: (0, i, 0)),
            pl.BlockSpec((_NWORK, br), lambda i: (0, i)),
            pl.BlockSpec((_DC, _DC), lambda i: (0, 0)),
            pl.BlockSpec((_DC, 8), lambda i: (0, 0)),
        ],
        out_specs=[
            pl.BlockSpec((br, _DC), lambda i: (i, 0)),
            pl.BlockSpec((br, _DC), lambda i: (i, 0)),
            pl.BlockSpec((br, 8), lambda i: (i, 0)),
        ],
        out_shape=[
            sds((_NUSER, _DC), _f32),
            sds((_NUSER, _DC), _f32),
            sds((_NUSER, 8), _f32),
        ],
    )(pref, outp, denp, Wg, a2)


def _fin_body(x, outp, denp, y_o, den_o):
    o = outp[0] + outp[1]
    ones = jnp.ones((denp.shape[0], 1), _f32)
    dn = lax.dot_general(denp[...], ones, (((0,), (0,)), ((), ())),
                         preferred_element_type=_f32)
    y_o[...] = x[...] + _leaky(o / (dn + 1e-16), 0.01)
    den_o[...] = jnp.sum(denp[...], axis=0, keepdims=True)


def _fin_call(x, outp, denp):
    br = 1024
    sds = jax.ShapeDtypeStruct
    return pl.pallas_call(
        _fin_body,
        grid=(10,),
        in_specs=[
            pl.BlockSpec((br, _DC), lambda i: (i, 0)),
            pl.BlockSpec((2, br, _DC), lambda i: (0, i, 0)),
            pl.BlockSpec((denp.shape[0], br), lambda i: (0, i)),
        ],
        out_specs=[
            pl.BlockSpec((br, _DC), lambda i: (i, 0)),
            pl.BlockSpec((1, br), lambda i: (0, i)),
        ],
        out_shape=[
            sds((_NN, _DC), _f32),
            sds((1, _NNP), _f32),
        ],
    )(x, outp, denp)


def _stu_body(raw, Ws, bs, Wt, bt, Wc, bc, lab, cls_o, kd_o, fl_o, tf_o):
    s = jnp.dot(raw[...], Ws[...], preferred_element_type=_f32) + bs[...]
    t = jnp.dot(raw[...], Wt[...], preferred_element_type=_f32) + bt[...]
    logits = jnp.dot(s, Wc[...], preferred_element_type=_f32) + bc[...]
    m = jnp.max(logits, axis=-1, keepdims=True)
    lse = m + jnp.log(jnp.sum(jnp.exp(logits - m), axis=-1, keepdims=True))
    logp = logits - lse
    oh = (lab[...] == lax.broadcasted_iota(_i32, (1, _NCLS), 1)).astype(_f32)
    cls_o[...] = -jnp.sum(oh * logp, axis=(0, 1), keepdims=True) / _BATCH
    kd_o[...] = (_TDECAY / (_BATCH * _DC)) * jnp.sum(
        (s - t) ** 2, axis=(0, 1), keepdims=True)
    fl_o[...] = jnp.sum((_l2n(s) - _l2n(t)) ** 2, axis=(0, 1),
                        keepdims=True) / (_BATCH * _DC)
    tf_o[...] = _leaky(s, 0.01)


def _stu_call(raw, Ws, bs2, Wt, bt2, Wc, bc2, lab):
    sds = jax.ShapeDtypeStruct
    return pl.pallas_call(
        _stu_body,
        out_shape=[
            sds((1, 1), _f32),
            sds((1, 1), _f32),
            sds((1, 1), _f32),
            sds((_BATCH, _DC), _f32),
        ],
    )(raw, Ws, bs2, Wt, bt2, Wc, bc2, lab)


def _bucket_call(srcf, dstf):
    epw = srcf.shape[1]
    ng = epw // 16
    nb = 6
    mesh = plsc.VectorSubcoreMesh(core_axis_name="c", subcore_axis_name="s")
    sds = jax.ShapeDtypeStruct

    def body(src_h, dst_h, srcc_o, dstc_o, st_o,
             src_v, dst_v, srcc_v, dstc_v, st_v):
        cid = lax.axis_index("c")
        sid = lax.axis_index("s")
        wid = sid * 2 + cid
        pltpu.sync_copy(src_h.at[wid], src_v)
        pltpu.sync_copy(dst_h.at[wid], dst_v)
        z = jnp.zeros((), _i32)

        @pl.loop(0, ng, init_carry=(z, z, z, z, z, z))
        def cnt(i, c):
            d16 = dst_v[pl.ds(i * 16, 16)]
            b16 = d16 // _TRASH
            return tuple(c[k] + jnp.sum((b16 == k).astype(_i32))
                         for k in range(nb))

        starts = [jnp.zeros((), _i32)]
        for k in range(nb - 1):
            starts.append(starts[-1] + cnt[k])
        iot = lax.iota(_i32, 16)
        stv = jnp.zeros((16,), _i32)
        for k in range(nb):
            stv = stv + jnp.where(iot == k, 1, 0) * starts[k]
        stv = stv + jnp.where(iot == nb, 1, 0) * epw
        st_v[pl.ds(0, 16)] = stv
        pltpu.sync_copy(st_v, st_o.at[wid])

        @pl.loop(0, ng, init_carry=tuple(starts))
        def _(i, off):
            s16 = src_v[pl.ds(i * 16, 16)]
            d16 = dst_v[pl.ds(i * 16, 16)]
            b16 = d16 // _TRASH
            new = []
            for k in range(nb):
                mk = b16 == k
                plsc.store_compressed(srcc_v.at[pl.ds(off[k], 16)], s16, mask=mk)
                plsc.store_compressed(dstc_v.at[pl.ds(off[k], 16)], d16, mask=mk)
                new.append(off[k] + jnp.sum(mk.astype(_i32)))
            return tuple(new)

        pltpu.sync_copy(srcc_v, srcc_o.at[wid])
        pltpu.sync_copy(dstc_v, dstc_o.at[wid])

    k = functools.partial(
        pl.kernel,
        out_type=(sds((_NWORK, epw + 16), _i32),
                  sds((_NWORK, epw + 16), _i32),
                  sds((_NWORK, 16), _i32)),
        mesh=mesh,
        compiler_params=pltpu.CompilerParams(needs_layout_passes=False),
        scratch_types=[
            pltpu.VMEM((epw,), _i32),
            pltpu.VMEM((epw,), _i32),
            pltpu.VMEM((epw + 16,), _i32),
            pltpu.VMEM((epw + 16,), _i32),
            pltpu.VMEM((16,), _i32),
        ],
    )(body)
    return k(srcf, dstf)


def _conv_call(npass, p0, s, d, h_arr, srcc, dstc, stc, zn, zacc):
    epw = srcc.shape[1]
    grp = _KCH // 16
    stripe = _ACCR // 16
    mesh = plsc.VectorSubcoreMesh(core_axis_name="c", subcore_axis_name="s")
    sds = jax.ShapeDtypeStruct

    def body(s_h, d_h, h_h, src_h, dst_h, st_h, zn_h, zacc_h,
             out_o, den_o, s_v, d_v, den_v, src_v, dst_v, st_v, d2b,
             rows_v, acc, sem0, sem1, ssem0, ssem1):
        cid = lax.axis_index("c")
        sid = lax.axis_index("s")
        wid = sid * 2 + cid
        pltpu.sync_copy(s_h, s_v)
        pltpu.sync_copy(d_h, d_v)
        pltpu.sync_copy(src_h.at[wid], src_v)
        pltpu.sync_copy(dst_h.at[wid], dst_v)
        pltpu.sync_copy(st_h.at[wid], st_v)
        pltpu.sync_copy(zn_h, den_v)
        pltpu.sync_copy(zacc_h, acc.at[pl.ds(sid * stripe, stripe)])
        plsc.subcore_barrier()
        iot = lax.iota(_i32, 16)
        sems = (sem0, sem1)
        ssems = (ssem0, ssem1)

        def issue(b, j):
            pltpu.async_copy(
                h_h.at[src_v.at[pl.ds(b * _KCH, _KCH)]], rows_v.at[j],
                sems[j])

        def wait(j):
            pltpu.make_async_copy(
                h_h.at[pl.ds(0, _KCH)], rows_v.at[j], sems[j]).wait()

        stv = st_v[pl.ds(0, 16)]
        for q in range(npass):
            p = p0 + q
            base = p * _TRASH
            e0 = stv[p]
            e1 = stv[p + 1] if p + 1 < 6 else jnp.int32(epw)
            c0 = e0 // _KCH
            c1 = (e1 + _KCH - 1) // _KCH

            issue(c0, 0)

            @pl.when(c0 + 1 < c1)
            def _():
                issue(c0 + 1, 1)

            @pl.loop(0, (c1 - c0 + 1) // 2)
            def _pair(i, _base=base, _e0=e0, _e1=e1, _c0=c0, _c1=c1):
                for jj in range(2):
                    b = _c0 + i * 2 + jj

                    @pl.when(b < _c1)
                    def _():
                        wait(jj)
                        for g in range(grp):
                            eo = b * _KCH + g * 16
                            src16 = src_v[pl.ds(eo, 16)]
                            dst16 = dst_v[pl.ds(eo, 16)]
                            se = plsc.load_gather(s_v, [src16])
                            de = plsc.load_gather(d_v, [dst16])
                            e = se + de
                            ex = jnp.exp(jnp.where(e >= 0, e, 0.2 * e))
                            gi = eo + iot
                            ex = jnp.where((gi >= _e0) & (gi < _e1), ex, 0.0)
                            plsc.addupdate_scatter(den_v, [dst16], ex)
                            r16 = dst16 - _base
                            r16 = jnp.where((r16 >= 0) & (r16 < _TRASH),
                                            r16, _TRASH)
                            d2b[jj, pl.ds(g * 16, 16)] = r16
                            for t in range(16):
                                w = _vbcast(ex, t)
                                r = g * 16 + t
                                for c in range(8):
                                    cs = pl.ds(c * 16, 16)
                                    rows_v[jj, r, cs] = rows_v[jj, r, cs] * w
                        pltpu.async_copy(rows_v.at[jj], acc.at[d2b.at[jj]],
                                         ssems[jj], add=True)

                for jj in range(2):
                    b = _c0 + i * 2 + jj

                    @pl.when(b < _c1)
                    def _():
                        pltpu.make_async_copy(h_h.at[pl.ds(0, _KCH)],
                                              rows_v.at[jj],
                                              ssems[jj]).wait()

                        @pl.when(b + 2 < _c1)
                        def _():
                            issue(b + 2, jj)

            plsc.subcore_barrier()
            pltpu.sync_copy(
                acc.at[pl.ds(sid * stripe, stripe)],
                out_o.at[cid, pl.ds(q * _ACCR + sid * stripe, stripe)])
            if q < npass - 1:
                pltpu.sync_copy(zacc_h,
                                acc.at[pl.ds(sid * stripe, stripe)])
                plsc.subcore_barrier()

        pltpu.sync_copy(den_v, den_o.at[wid])

    k = functools.partial(
        pl.kernel,
        out_type=(sds((2, npass * _ACCR, _DC), _f32),
                  sds((_NWORK, _NNP), _f32)),
        mesh=mesh,
        compiler_params=pltpu.CompilerParams(needs_layout_passes=False),
        scratch_types=[
            pltpu.VMEM((_NNP,), _f32),
            pltpu.VMEM((_NNP,), _f32),
            pltpu.VMEM((_NNP,), _f32),
            pltpu.VMEM((epw,), _i32),
            pltpu.VMEM((epw,), _i32),
            pltpu.VMEM((16,), _i32),
            pltpu.VMEM((8, _KCH), _i32),
            pltpu.VMEM((2, _KCH, _DC), _f32),
            pltpu.VMEM_SHARED((_ACCR, _DC), _f32),
            pltpu.SemaphoreType.DMA,
            pltpu.SemaphoreType.DMA,
            pltpu.SemaphoreType.DMA,
            pltpu.SemaphoreType.DMA,
        ],
    )(body)
    return k(s, d, h_arr, srcc, dstc, stc, zn, zacc)


def _alpha_call(s, d, den, srcf, dstf):
    epw = srcf.shape[1]
    nch = epw // _KCH
    grp = _KCH // 16
    mesh = plsc.VectorSubcoreMesh(core_axis_name="c", subcore_axis_name="s")
    sds = jax.ShapeDtypeStruct

    def body(s_h, d_h, den_h, src_h, dst_h, al_o,
             s_v, d_v, den_v, src_v, dst_v, al_v):
        cid = lax.axis_index("c")
        sid = lax.axis_index("s")
        wid = sid * 2 + cid
        pltpu.sync_copy(s_h, s_v)
        pltpu.sync_copy(d_h, d_v)
        pltpu.sync_copy(den_h, den_v)
        pltpu.sync_copy(src_h.at[wid], src_v)
        pltpu.sync_copy(dst_h.at[wid], dst_v)

        @pl.loop(0, nch)
        def _chunk(b):
            for g in range(grp):
                eo = b * _KCH + g * 16
                src16 = src_v[pl.ds(eo, 16)]
                dst16 = dst_v[pl.ds(eo, 16)]
                se = plsc.load_gather(s_v, [src16])
                de = plsc.load_gather(d_v, [dst16])
                e = se + de
                ex = jnp.exp(jnp.where(e >= 0, e, 0.2 * e))
                dn = plsc.load_gather(den_v, [dst16])
                al_v[pl.ds(eo, 16)] = ex / (dn + 1e-16)

        pltpu.sync_copy(al_v, al_o.at[wid])

    k = functools.partial(
        pl.kernel,
        out_type=sds((_NWORK, epw), _f32),
        mesh=mesh,
        compiler_params=pltpu.CompilerParams(needs_layout_passes=False),
        scratch_types=[
            pltpu.VMEM((_NNP,), _f32),
            pltpu.VMEM((_NNP,), _f32),
            pltpu.VMEM((_NNP,), _f32),
            pltpu.VMEM((epw,), _i32),
            pltpu.VMEM((epw,), _i32),
            pltpu.VMEM((epw,), _f32),
        ],
    )(body)
    return k(s, d, den, srcf, dstf)


def _stu_gather_call(features, idxr, meta_label):
    bpw = _BATCH // _NWORK
    mesh = plsc.VectorSubcoreMesh(core_axis_name="c", subcore_axis_name="s")
    sds = jax.ShapeDtypeStruct

    def body(feat_h, idx_h, ml_h, raw_o, lab_o, idx_v, rows_v, ml_v, lab_v, sem):
        cid = lax.axis_index("c")
        sid = lax.axis_index("s")
        wid = sid * 2 + cid
        pltpu.sync_copy(idx_h.at[wid], idx_v)
        pltpu.async_copy(feat_h.at[idx_v], rows_v, sem).wait()
        pltpu.sync_copy(rows_v, raw_o.at[pl.ds(wid * bpw, bpw)])
        pltpu.sync_copy(ml_h, ml_v)
        for g in range(bpw // 16):
            ii = idx_v[pl.ds(g * 16, 16)]
            lab_v[pl.ds(g * 16, 16)] = plsc.load_gather(ml_v, [ii])
        pltpu.sync_copy(lab_v, lab_o.at[wid])

    k = functools.partial(
        pl.kernel,
        out_type=(sds((_BATCH, _DF), _f32), sds((_NWORK, bpw), _i32)),
        mesh=mesh,
        compiler_params=pltpu.CompilerParams(needs_layout_passes=False),
        scratch_types=[
            pltpu.VMEM((bpw,), _i32),
            pltpu.VMEM((bpw, _DF), _f32),
            pltpu.VMEM((_NITEM,), _i32),
            pltpu.VMEM((bpw,), _i32),
            pltpu.SemaphoreType.DMA,
        ],
    )(body)
    return k(features, idxr, meta_label)


def _pack_edges(src, dst):
    e = src.shape[0]
    epw = -(-e // (_NWORK * _KCH)) * _KCH
    tot = epw * _NWORK
    if tot > e:
        pad = jnp.zeros((tot - e,), _i32)
        src = jnp.concatenate([src, pad])
        dst = jnp.concatenate([dst, pad])
    return src.reshape(_NWORK, epw), dst.reshape(_NWORK, epw)


def kernel(features, preference, W_gat, att_src, att_dst, W_s, b_s, W_t, b_t,
           W_c, b_c, edge_index, item_tensor, meta_label):
    a2 = jnp.pad(jnp.stack([att_src, att_dst], axis=1), ((0, 0), (0, 6)))
    bs2 = b_s.reshape(1, -1)
    bt2 = b_t.reshape(1, -1)
    bc2 = b_c.reshape(1, -1)

    f_n, pref, h_it, h_u, sd_it, sd_u = _pre_call(
        features, W_s, bs2, preference, W_gat, a2)

    src0 = edge_index[0]
    dst0 = edge_index[1]
    srcR, dstR = _pack_edges(src0, dst0)
    src1 = jnp.concatenate([src0, dst0])
    dst1 = jnp.concatenate([dst0, src0])
    srcF, dstF = _pack_edges(src1, dst1)

    srcRc, dstRc, stR = _bucket_call(srcR, dstR)
    srcRc, dstRc = srcRc[:, :-16], dstRc[:, :-16]
    srcFc, dstFc, stF = _bucket_call(srcF, dstF)
    srcFc, dstFc = srcFc[:, :-16], dstFc[:, :-16]

    zn = jnp.zeros((_NNP,), _f32)
    zacc = jnp.zeros((_ACCR // 16, _DC), _f32)
    zpad = jnp.zeros((_NNP - _NN,), _f32)

    for _ in range(_NROUT):
        h = jnp.concatenate([h_u, h_it], axis=0)
        s = jnp.concatenate([sd_u[:, 0], sd_it[:, 0], zpad])
        d = jnp.concatenate([sd_u[:, 1], sd_it[:, 1], zpad])
        out, denp = _conv_call(_NPASS_R, 0, s, d, h, srcRc, dstRc, stR,
                               zn, zacc)
        outp = jnp.concatenate(
            [out[:, p * _ACCR: p * _ACCR + _TRASH] for p in range(_NPASS_R)],
            axis=1)
        pref, h_u, sd_u = _rout_call(pref, outp, denp, W_gat, a2)

    h = jnp.concatenate([h_u, h_it], axis=0)
    s = jnp.concatenate([sd_u[:, 0], sd_it[:, 0], zpad])
    d = jnp.concatenate([sd_u[:, 1], sd_it[:, 1], zpad])
    outA, denpA = _conv_call(3, 0, s, d, h, srcFc, dstFc, stF, zn, zacc)
    outB, denpB = _conv_call(3, 3, s, d, h, srcFc, dstFc, stF, zn, zacc)
    denp = jnp.concatenate([denpA, denpB], axis=0)
    outp = jnp.concatenate(
        [outA[:, p * _ACCR: p * _ACCR + _TRASH] for p in range(3)]
        + [outB[:, p * _ACCR: p * _ACCR + _TRASH] for p in range(3)],
        axis=1)
    x = jnp.concatenate([pref, f_n], axis=0)
    y, den_sum = _fin_call(x, outp, denp)
    alpha = _alpha_call(s, d, den_sum.reshape(-1), srcF, dstF)
    alpha = alpha.reshape(-1)[: src1.shape[0]].reshape(-1, 1)

    item_nodes = item_tensor - _NUSER
    raw, labr = _stu_gather_call(features, item_nodes.reshape(_NWORK, -1),
                                 meta_label)
    lab = labr.reshape(-1, 1)
    cls, kd, fl, tf = _stu_call(raw, W_s, bs2, W_t, bt2, W_c, bc2, lab)

    return (y, alpha, cls.reshape(()), kd.reshape(()), fl.reshape(()), tf)


# speedup vs baseline: 1.1103x; 1.1103x over previous
import functools

import jax
import jax.numpy as jnp
from jax import lax
from jax.experimental import pallas as pl
from jax.experimental.pallas import tpu as pltpu
from jax.experimental.pallas import tpu_sc as plsc

_NUSER = 5000
_NITEM = 5000
_NN = 10000
_NNP = 10240
_DC = 128
_DF = 512
_NCLS = 32
_NROUT = 3
_TDECAY = 0.5
_BATCH = 1024

_NWORK = 32
_ACCR = 1920
_TRASH = 1904
_KCH = 80
_STRIPE = _NNP // 16

_NPASS_R = -(-_NUSER // _TRASH)
_NPASS_F = -(-_NN // _TRASH)

_f32 = jnp.float32
_i32 = jnp.int32


def _l2n(x):
    return x / jnp.maximum(jnp.sqrt(jnp.sum(x * x, axis=-1, keepdims=True)), 1e-12)


def _leaky(x, s):
    return jnp.where(x >= 0, x, s * x)


_GD = lax.GatherDimensionNumbers(offset_dims=(), collapsed_slice_dims=(0,),
                                 start_index_map=(0,))


def _vbcast(v, t):
    idx = jnp.full((16, 1), t, _i32)
    return lax.gather(v, idx, _GD, slice_sizes=(1,),
                      mode=lax.GatherScatterMode.PROMISE_IN_BOUNDS)


def _pre_body(feat, Ws, bs, prefr, Wg, a2, f_o, pref_o, hi_o, hu_o, sdi_o, sdu_o):
    ft = jnp.dot(feat[...], Ws[...], preferred_element_type=_f32) + bs[...]
    f = _l2n(_leaky(ft, 0.01))
    p = _l2n(prefr[...])
    hi = jnp.dot(f, Wg[...], preferred_element_type=_f32)
    hu = jnp.dot(p, Wg[...], preferred_element_type=_f32)
    f_o[...] = f
    pref_o[...] = p
    hi_o[...] = hi
    hu_o[...] = hu
    sdi_o[...] = jnp.dot(hi, a2[...], preferred_element_type=_f32)
    sdu_o[...] = jnp.dot(hu, a2[...], preferred_element_type=_f32)


def _pre_call(features, Ws, bs2, preference, Wg, a2):
    br = 1024
    sds = jax.ShapeDtypeStruct
    return pl.pallas_call(
        _pre_body,
        grid=(5,),
        in_specs=[
            pl.BlockSpec((br, _DF), lambda i: (i, 0)),
            pl.BlockSpec((_DF, _DC), lambda i: (0, 0)),
            pl.BlockSpec((1, _DC), lambda i: (0, 0)),
            pl.BlockSpec((br, _DC), lambda i: (i, 0)),
            pl.BlockSpec((_DC, _DC), lambda i: (0, 0)),
            pl.BlockSpec((_DC, 8), lambda i: (0, 0)),
        ],
        out_specs=[
            pl.BlockSpec((br, _DC), lambda i: (i, 0)),
            pl.BlockSpec((br, _DC), lambda i: (i, 0)),
            pl.BlockSpec((br, _DC), lambda i: (i, 0)),
            pl.BlockSpec((br, _DC), lambda i: (i, 0)),
            pl.BlockSpec((br, 8), lambda i: (i, 0)),
            pl.BlockSpec((br, 8), lambda i: (i, 0)),
        ],
        out_shape=[
            sds((_NITEM, _DC), _f32),
            sds((_NUSER, _DC), _f32),
            sds((_NITEM, _DC), _f32),
            sds((_NUSER, _DC), _f32),
            sds((_NITEM, 8), _f32),
            sds((_NUSER, 8), _f32),
        ],
    )(features, Ws, bs2, preference, Wg, a2)


def _rout_body(pref, outp, denp, Wg, a2, pref_o, hu_o, sdu_o):
    o = outp[0] + outp[1]
    ones = jnp.ones((denp.shape[0], 1), _f32)
    dn = lax.dot_general(denp[...], ones, (((0,), (0,)), ((), ())),
                         preferred_element_type=_f32)
    p = _l2n(pref[...] + o / (dn + 1e-16))
    hu = jnp.dot(p, Wg[...], preferred_element_type=_f32)
    pref_o[...] = p
    hu_o[...] = hu
    sdu_o[...] = jnp.dot(hu, a2[...], preferred_element_type=_f32)


def _rout_call(pref, outp, denp, Wg, a2):
    br = 1024
    sds = jax.ShapeDtypeStruct
    return pl.pallas_call(
        _rout_body,
        grid=(5,),
        in_specs=[
            pl.BlockSpec((br, _DC), lambda i: (i, 0)),
            pl.BlockSpec((2, br, _DC), lambda i: (0, i, 0)),
            pl.BlockSpec((_NWORK, br), lambda i: (0, i)),
            pl.BlockSpec((_DC, _DC), lambda i: (0, 0)),
            pl.BlockSpec((_DC, 8), lambda i: (0, 0)),
        ],
        out_specs=[
            pl.BlockSpec((br, _DC), lambda i: (i, 0)),
            pl.BlockSpec((br, _DC), lambda i: (i, 0)),
            pl.BlockSpec((br, 8), lambda i: (i, 0)),
        ],
        out_shape=[
            sds((_NUSER, _DC), _f32),
            sds((_NUSER, _DC), _f32),
            sds((_NUSER, 8), _f32),
        ],
    )(pref, outp, denp, Wg, a2)


def _fin_body(x, outp, denp, y_o, den_o):
    o = outp[0] + outp[1]
    ones = jnp.ones((denp.shape[0], 1), _f32)
    dn = lax.dot_general(denp[...], ones, (((0,), (0,)), ((), ())),
                         preferred_element_type=_f32)
    y_o[...] = x[...] + _leaky(o / (dn + 1e-16), 0.01)
    den_o[...] = jnp.sum(denp[...], axis=0, keepdims=True)


def _fin_call(x, outp, denp):
    br = 1024
    sds = jax.ShapeDtypeStruct
    return pl.pallas_call(
        _fin_body,
        grid=(10,),
        in_specs=[
            pl.BlockSpec((br, _DC), lambda i: (i, 0)),
            pl.BlockSpec((2, br, _DC), lambda i: (0, i, 0)),
            pl.BlockSpec((denp.shape[0], br), lambda i: (0, i)),
        ],
        out_specs=[
            pl.BlockSpec((br, _DC), lambda i: (i, 0)),
            pl.BlockSpec((1, br), lambda i: (0, i)),
        ],
        out_shape=[
            sds((_NN, _DC), _f32),
            sds((1, _NNP), _f32),
        ],
    )(x, outp, denp)


def _stu_body(raw, Ws, bs, Wt, bt, Wc, bc, lab, cls_o, kd_o, fl_o, tf_o):
    s = jnp.dot(raw[...], Ws[...], preferred_element_type=_f32) + bs[...]
    t = jnp.dot(raw[...], Wt[...], preferred_element_type=_f32) + bt[...]
    logits = jnp.dot(s, Wc[...], preferred_element_type=_f32) + bc[...]
    m = jnp.max(logits, axis=-1, keepdims=True)
    lse = m + jnp.log(jnp.sum(jnp.exp(logits - m), axis=-1, keepdims=True))
    logp = logits - lse
    oh = (lab[...] == lax.broadcasted_iota(_i32, (1, _NCLS), 1)).astype(_f32)
    cls_o[...] = -jnp.sum(oh * logp, axis=(0, 1), keepdims=True) / _BATCH
    kd_o[...] = (_TDECAY / (_BATCH * _DC)) * jnp.sum(
        (s - t) ** 2, axis=(0, 1), keepdims=True)
    fl_o[...] = jnp.sum((_l2n(s) - _l2n(t)) ** 2, axis=(0, 1),
                        keepdims=True) / (_BATCH * _DC)
    tf_o[...] = _leaky(s, 0.01)


def _stu_call(raw, Ws, bs2, Wt, bt2, Wc, bc2, lab):
    sds = jax.ShapeDtypeStruct
    return pl.pallas_call(
        _stu_body,
        out_shape=[
            sds((1, 1), _f32),
            sds((1, 1), _f32),
            sds((1, 1), _f32),
            sds((_BATCH, _DC), _f32),
        ],
    )(raw, Ws, bs2, Wt, bt2, Wc, bc2, lab)


def _bucket_call(srcf, dstf):
    epw = srcf.shape[1]
    ng = epw // 16
    nb = 6
    mesh = plsc.VectorSubcoreMesh(core_axis_name="c", subcore_axis_name="s")
    sds = jax.ShapeDtypeStruct

    def body(src_h, dst_h, srcc_o, dstc_o, st_o,
             src_v, dst_v, srcc_v, dstc_v, st_v):
        cid = lax.axis_index("c")
        sid = lax.axis_index("s")
        wid = sid * 2 + cid
        pltpu.sync_copy(src_h.at[wid], src_v)
        pltpu.sync_copy(dst_h.at[wid], dst_v)
        z = jnp.zeros((), _i32)

        @pl.loop(0, ng, init_carry=(z, z, z, z, z, z))
        def cnt(i, c):
            d16 = dst_v[pl.ds(i * 16, 16)]
            b16 = d16 // _TRASH
            return tuple(c[k] + jnp.sum((b16 == k).astype(_i32))
                         for k in range(nb))

        starts = [jnp.zeros((), _i32)]
        for k in range(nb - 1):
            starts.append(starts[-1] + cnt[k])
        iot = lax.iota(_i32, 16)
        stv = jnp.zeros((16,), _i32)
        for k in range(nb):
            stv = stv + jnp.where(iot == k, 1, 0) * starts[k]
        stv = stv + jnp.where(iot == nb, 1, 0) * epw
        st_v[pl.ds(0, 16)] = stv
        pltpu.sync_copy(st_v, st_o.at[wid])

        @pl.loop(0, ng, init_carry=tuple(starts))
        def _(i, off):
            s16 = src_v[pl.ds(i * 16, 16)]
            d16 = dst_v[pl.ds(i * 16, 16)]
            b16 = d16 // _TRASH
            new = []
            for k in range(nb):
                mk = b16 == k
                plsc.store_compressed(srcc_v.at[pl.ds(off[k], 16)], s16, mask=mk)
                plsc.store_compressed(dstc_v.at[pl.ds(off[k], 16)], d16, mask=mk)
                new.append(off[k] + jnp.sum(mk.astype(_i32)))
            return tuple(new)

        pltpu.sync_copy(srcc_v, srcc_o.at[wid])
        pltpu.sync_copy(dstc_v, dstc_o.at[wid])

    k = functools.partial(
        pl.kernel,
        out_type=(sds((_NWORK, epw + 16), _i32),
                  sds((_NWORK, epw + 16), _i32),
                  sds((_NWORK, 16), _i32)),
        mesh=mesh,
        compiler_params=pltpu.CompilerParams(needs_layout_passes=False),
        scratch_types=[
            pltpu.VMEM((epw,), _i32),
            pltpu.VMEM((epw,), _i32),
            pltpu.VMEM((epw + 16,), _i32),
            pltpu.VMEM((epw + 16,), _i32),
            pltpu.VMEM((16,), _i32),
        ],
    )(body)
    return k(srcf, dstf)


def _conv_call(npass, p0, s, d, h_arr, srcc, dstc, stc, zn, zacc):
    epw = srcc.shape[1]
    grp = _KCH // 16
    stripe = _ACCR // 16
    mesh = plsc.VectorSubcoreMesh(core_axis_name="c", subcore_axis_name="s")
    sds = jax.ShapeDtypeStruct

    def body(s_h, d_h, h_h, src_h, dst_h, st_h, zn_h, zacc_h,
             out_o, den_o, s_v, d_v, den_v, src_v, dst_v, st_v, d2b,
             rows_v, acc, sem0, sem1):
        cid = lax.axis_index("c")
        sid = lax.axis_index("s")
        wid = sid * 2 + cid
        pltpu.sync_copy(s_h, s_v)
        pltpu.sync_copy(d_h, d_v)
        pltpu.sync_copy(src_h.at[wid], src_v)
        pltpu.sync_copy(dst_h.at[wid], dst_v)
        pltpu.sync_copy(st_h.at[wid], st_v)
        pltpu.sync_copy(zn_h, den_v)
        pltpu.sync_copy(zacc_h, acc.at[pl.ds(sid * stripe, stripe)])
        plsc.subcore_barrier()
        iot = lax.iota(_i32, 16)
        sems = (sem0, sem1)

        def issue(b, j):
            pltpu.async_copy(
                h_h.at[src_v.at[pl.ds(b * _KCH, _KCH)]], rows_v.at[j],
                sems[j])

        def wait(j):
            pltpu.make_async_copy(
                h_h.at[pl.ds(0, _KCH)], rows_v.at[j], sems[j]).wait()

        stv = st_v[pl.ds(0, 16)]
        for q in range(npass):
            p = p0 + q
            base = p * _TRASH
            e0 = stv[p]
            e1 = stv[p + 1] if p + 1 < 6 else jnp.int32(epw)
            c0 = e0 // _KCH
            c1 = (e1 + _KCH - 1) // _KCH

            issue(c0, 0)

            @pl.when(c0 + 1 < c1)
            def _():
                issue(c0 + 1, 1)

            @pl.loop(0, (c1 - c0 + 1) // 2)
            def _pair(i, _base=base, _e0=e0, _e1=e1, _c0=c0, _c1=c1):
                for jj in range(2):
                    b = _c0 + i * 2 + jj

                    @pl.when(b < _c1)
                    def _():
                        wait(jj)
                        for g in range(grp):
                            eo = b * _KCH + g * 16
                            src16 = src_v[pl.ds(eo, 16)]
                            dst16 = dst_v[pl.ds(eo, 16)]
                            se = plsc.load_gather(s_v, [src16])
                            de = plsc.load_gather(d_v, [dst16])
                            e = se + de
                            ex = jnp.exp(jnp.where(e >= 0, e, 0.2 * e))
                            gi = eo + iot
                            ex = jnp.where((gi >= _e0) & (gi < _e1), ex, 0.0)
                            plsc.addupdate_scatter(den_v, [dst16], ex)
                            r16 = dst16 - _base
                            r16 = jnp.where((r16 >= 0) & (r16 < _TRASH),
                                            r16, _TRASH)
                            d2b[jj, pl.ds(g * 16, 16)] = r16
                            for t in range(16):
                                w = _vbcast(ex, t)
                                r = g * 16 + t
                                for c in range(8):
                                    cs = pl.ds(c * 16, 16)
                                    rows_v[jj, r, cs] = rows_v[jj, r, cs] * w
                        pltpu.sync_copy(rows_v.at[jj], acc.at[d2b.at[jj]],
                                        add=True)

                        @pl.when(b + 2 < _c1)
                        def _():
                            issue(b + 2, jj)

            plsc.subcore_barrier()
            pltpu.sync_copy(
                acc.at[pl.ds(sid * stripe, stripe)],
                out_o.at[cid, pl.ds(q * _ACCR + sid * stripe, stripe)])
            if q < npass - 1:
                pltpu.sync_copy(zacc_h,
                                acc.at[pl.ds(sid * stripe, stripe)])
                plsc.subcore_barrier()

        pltpu.sync_copy(den_v, den_o.at[wid])

    k = functools.partial(
        pl.kernel,
        out_type=(sds((2, npass * _ACCR, _DC), _f32),
                  sds((_NWORK, _NNP), _f32)),
        mesh=mesh,
        compiler_params=pltpu.CompilerParams(needs_layout_passes=False),
        scratch_types=[
            pltpu.VMEM((_NNP,), _f32),
            pltpu.VMEM((_NNP,), _f32),
            pltpu.VMEM((_NNP,), _f32),
            pltpu.VMEM((epw,), _i32),
            pltpu.VMEM((epw,), _i32),
            pltpu.VMEM((16,), _i32),
            pltpu.VMEM((8, _KCH), _i32),
            pltpu.VMEM((2, _KCH, _DC), _f32),
            pltpu.VMEM_SHARED((_ACCR, _DC), _f32),
            pltpu.SemaphoreType.DMA,
            pltpu.SemaphoreType.DMA,
        ],
    )(body)
    return k(s, d, h_arr, srcc, dstc, stc, zn, zacc)


def _alpha_call(s, d, den, srcf, dstf):
    epw = srcf.shape[1]
    nch = epw // _KCH
    grp = _KCH // 16
    mesh = plsc.VectorSubcoreMesh(core_axis_name="c", subcore_axis_name="s")
    sds = jax.ShapeDtypeStruct

    def body(s_h, d_h, den_h, src_h, dst_h, al_o,
             s_v, d_v, den_v, src_v, dst_v, al_v):
        cid = lax.axis_index("c")
        sid = lax.axis_index("s")
        wid = sid * 2 + cid
        pltpu.sync_copy(s_h, s_v)
        pltpu.sync_copy(d_h, d_v)
        pltpu.sync_copy(den_h, den_v)
        pltpu.sync_copy(src_h.at[wid], src_v)
        pltpu.sync_copy(dst_h.at[wid], dst_v)

        @pl.loop(0, nch)
        def _chunk(b):
            for g in range(grp):
                eo = b * _KCH + g * 16
                src16 = src_v[pl.ds(eo, 16)]
                dst16 = dst_v[pl.ds(eo, 16)]
                se = plsc.load_gather(s_v, [src16])
                de = plsc.load_gather(d_v, [dst16])
                e = se + de
                ex = jnp.exp(jnp.where(e >= 0, e, 0.2 * e))
                dn = plsc.load_gather(den_v, [dst16])
                al_v[pl.ds(eo, 16)] = ex / (dn + 1e-16)

        pltpu.sync_copy(al_v, al_o.at[wid])

    k = functools.partial(
        pl.kernel,
        out_type=sds((_NWORK, epw), _f32),
        mesh=mesh,
        compiler_params=pltpu.CompilerParams(needs_layout_passes=False),
        scratch_types=[
            pltpu.VMEM((_NNP,), _f32),
            pltpu.VMEM((_NNP,), _f32),
            pltpu.VMEM((_NNP,), _f32),
            pltpu.VMEM((epw,), _i32),
            pltpu.VMEM((epw,), _i32),
            pltpu.VMEM((epw,), _f32),
        ],
    )(body)
    return k(s, d, den, srcf, dstf)


def _stu_gather_call(features, idxr, meta_label):
    bpw = _BATCH // _NWORK
    mesh = plsc.VectorSubcoreMesh(core_axis_name="c", subcore_axis_name="s")
    sds = jax.ShapeDtypeStruct

    def body(feat_h, idx_h, ml_h, raw_o, lab_o, idx_v, rows_v, ml_v, lab_v, sem):
        cid = lax.axis_index("c")
        sid = lax.axis_index("s")
        wid = sid * 2 + cid
        pltpu.sync_copy(idx_h.at[wid], idx_v)
        pltpu.async_copy(feat_h.at[idx_v], rows_v, sem).wait()
        pltpu.sync_copy(rows_v, raw_o.at[pl.ds(wid * bpw, bpw)])
        pltpu.sync_copy(ml_h, ml_v)
        for g in range(bpw // 16):
            ii = idx_v[pl.ds(g * 16, 16)]
            lab_v[pl.ds(g * 16, 16)] = plsc.load_gather(ml_v, [ii])
        pltpu.sync_copy(lab_v, lab_o.at[wid])

    k = functools.partial(
        pl.kernel,
        out_type=(sds((_BATCH, _DF), _f32), sds((_NWORK, bpw), _i32)),
        mesh=mesh,
        compiler_params=pltpu.CompilerParams(needs_layout_passes=False),
        scratch_types=[
            pltpu.VMEM((bpw,), _i32),
            pltpu.VMEM((bpw, _DF), _f32),
            pltpu.VMEM((_NITEM,), _i32),
            pltpu.VMEM((bpw,), _i32),
            pltpu.SemaphoreType.DMA,
        ],
    )(body)
    return k(features, idxr, meta_label)


def _pack_edges(src, dst):
    e = src.shape[0]
    epw = -(-e // (_NWORK * _KCH)) * _KCH
    tot = epw * _NWORK
    if tot > e:
        pad = jnp.zeros((tot - e,), _i32)
        src = jnp.concatenate([src, pad])
        dst = jnp.concatenate([dst, pad])
    return src.reshape(_NWORK, epw), dst.reshape(_NWORK, epw)


def kernel(features, preference, W_gat, att_src, att_dst, W_s, b_s, W_t, b_t,
           W_c, b_c, edge_index, item_tensor, meta_label):
    a2 = jnp.pad(jnp.stack([att_src, att_dst], axis=1), ((0, 0), (0, 6)))
    bs2 = b_s.reshape(1, -1)
    bt2 = b_t.reshape(1, -1)
    bc2 = b_c.reshape(1, -1)

    f_n, pref, h_it, h_u, sd_it, sd_u = _pre_call(
        features, W_s, bs2, preference, W_gat, a2)

    src0 = edge_index[0]
    dst0 = edge_index[1]
    srcR, dstR = _pack_edges(src0, dst0)
    src1 = jnp.concatenate([src0, dst0])
    dst1 = jnp.concatenate([dst0, src0])
    srcF, dstF = _pack_edges(src1, dst1)

    srcRc, dstRc, stR = _bucket_call(srcR, dstR)
    srcRc, dstRc = srcRc[:, :-16], dstRc[:, :-16]
    srcFc, dstFc, stF = _bucket_call(srcF, dstF)
    srcFc, dstFc = srcFc[:, :-16], dstFc[:, :-16]

    zn = jnp.zeros((_NNP,), _f32)
    zacc = jnp.zeros((_ACCR // 16, _DC), _f32)
    zpad = jnp.zeros((_NNP - _NN,), _f32)

    for _ in range(_NROUT):
        h = jnp.concatenate([h_u, h_it], axis=0)
        s = jnp.concatenate([sd_u[:, 0], sd_it[:, 0], zpad])
        d = jnp.concatenate([sd_u[:, 1], sd_it[:, 1], zpad])
        out, denp = _conv_call(_NPASS_R, 0, s, d, h, srcRc, dstRc, stR,
                               zn, zacc)
        outp = jnp.concatenate(
            [out[:, p * _ACCR: p * _ACCR + _TRASH] for p in range(_NPASS_R)],
            axis=1)
        pref, h_u, sd_u = _rout_call(pref, outp, denp, W_gat, a2)

    h = jnp.concatenate([h_u, h_it], axis=0)
    s = jnp.concatenate([sd_u[:, 0], sd_it[:, 0], zpad])
    d = jnp.concatenate([sd_u[:, 1], sd_it[:, 1], zpad])
    outA, denpA = _conv_call(3, 0, s, d, h, srcFc, dstFc, stF, zn, zacc)
    outB, denpB = _conv_call(3, 3, s, d, h, srcFc, dstFc, stF, zn, zacc)
    denp = jnp.concatenate([denpA, denpB], axis=0)
    outp = jnp.concatenate(
        [outA[:, p * _ACCR: p * _ACCR + _TRASH] for p in range(3)]
        + [outB[:, p * _ACCR: p * _ACCR + _TRASH] for p in range(3)],
        axis=1)
    x = jnp.concatenate([pref, f_n], axis=0)
    y, den_sum = _fin_call(x, outp, denp)
    alpha = _alpha_call(s, d, den_sum.reshape(-1), srcF, dstF)
    alpha = alpha.reshape(-1)[: src1.shape[0]].reshape(-1, 1)

    item_nodes = item_tensor - _NUSER
    raw, labr = _stu_gather_call(features, item_nodes.reshape(_NWORK, -1),
                                 meta_label)
    lab = labr.reshape(-1, 1)
    cls, kd, fl, tf = _stu_call(raw, W_s, bs2, W_t, bt2, W_c, bc2, lab)

    return (y, alpha, cls.reshape(()), kd.reshape(()), fl.reshape(()), tf)
